# Initial kernel scaffold; baseline (speedup 1.0000x reference)
import jax
import jax.numpy as jnp
from jax.experimental import pallas as pl


def kernel(coords, h, flow_dir, params):
    raise NotImplementedError("write your pallas kernel here")


# speedup vs baseline: 3.0197x; 3.0197x over previous
import functools

import jax
import jax.numpy as jnp
from jax.experimental import pallas as pl
from jax.experimental.pallas import tpu as pltpu
from jax.experimental.pallas import tpu_sc as plsc

HID = 128
MSG = 256
MAXK = 32
R2 = 0.01
_INF = float('inf')
_SQRT_HALF = 0.7071067811865476


def _gelu(x):
    return 0.5 * x * (1.0 + jax.lax.erf(x * _SQRT_HALF))


def _sigmoid(x):
    return 1.0 / (1.0 + jnp.exp(-x))


def _pre_body(h_ref, x_ref, u_ref, lng_ref, lnb_ref, a_ref, b_ref, e0b_ref,
              gi_ref, gj_ref, feat_ref, q_ref, p_ref, gsum_ref, *, blk, n_real):
    pid = pl.program_id(0)
    hb = h_ref[...]
    mean = jnp.mean(hb, axis=1, keepdims=True)
    var = jnp.mean((hb - mean) * (hb - mean), axis=1, keepdims=True)
    feat = (hb - mean) / jnp.sqrt(var + 1e-5) * lng_ref[...] + lnb_ref[...]
    feat_ref[...] = feat
    s = jnp.sum(x_ref[...] * u_ref[...], axis=1, keepdims=True)
    q_ref[...] = (jnp.dot(feat, a_ref[...], preferred_element_type=jnp.float32)
                  + e0b_ref[...] + s * gi_ref[...])
    p_ref[...] = (jnp.dot(feat, b_ref[...], preferred_element_type=jnp.float32)
                  + s * gj_ref[...])
    rows = pid * blk + jax.lax.broadcasted_iota(jnp.int32, (blk, 1), 0)
    part = jnp.sum(jnp.where(rows < n_real, feat, 0.0), axis=0, keepdims=True)

    @pl.when(pid == 0)
    def _():
        gsum_ref[...] = jnp.zeros_like(gsum_ref)

    gsum_ref[...] += part


def _precompute(h_pad, x_pad, u_pad, lng, lnb, a_w, b_w, e0b, gi, gj, n_real):
    npad = h_pad.shape[0]
    blk = 128
    grid = npad // blk
    row_spec = lambda w: pl.BlockSpec((blk, w), lambda i: (i, 0))
    full_spec = lambda r, w: pl.BlockSpec((r, w), lambda i: (0, 0))
    return pl.pallas_call(
        functools.partial(_pre_body, blk=blk, n_real=n_real),
        grid=(grid,),
        in_specs=[row_spec(HID), row_spec(HID), full_spec(1, HID),
                  full_spec(1, HID), full_spec(1, HID),
                  full_spec(HID, MSG), full_spec(HID, MSG), full_spec(1, MSG),
                  full_spec(1, MSG), full_spec(1, MSG)],
        out_specs=[row_spec(HID), row_spec(MSG), row_spec(MSG),
                   full_spec(1, HID)],
        out_shape=[jax.ShapeDtypeStruct((npad, HID), jnp.float32),
                   jax.ShapeDtypeStruct((npad, MSG), jnp.float32),
                   jax.ShapeDtypeStruct((npad, MSG), jnp.float32),
                   jax.ShapeDtypeStruct((1, HID), jnp.float32)],
    )(h_pad, x_pad, u_pad, lng, lnb, a_w, b_w, e0b, gi, gj)


def _glob_body(gsum_ref, w0, b0, w1, b1, w2, b2, u0c, u0b, out_ref, *, n_real):
    g = gsum_ref[...] * (1.0 / n_real)
    g = _gelu(jnp.dot(g, w0[...], preferred_element_type=jnp.float32) + b0[...])
    g = _gelu(jnp.dot(g, w1[...], preferred_element_type=jnp.float32) + b1[...])
    g = jnp.dot(g, w2[...], preferred_element_type=jnp.float32) + b2[...]
    out_ref[...] = jnp.dot(g, u0c[...], preferred_element_type=jnp.float32) + u0b[...]


def _global_ctx(gsum, w0, b0, w1, b1, w2, b2, u0c, u0b, n_real):
    return pl.pallas_call(
        functools.partial(_glob_body, n_real=n_real),
        out_shape=jax.ShapeDtypeStruct((1, MSG), jnp.float32),
    )(gsum, w0, b0, w1, b1, w2, b2, u0c, u0b)


def _graph_body(x_ref, xt_ref, col_ref, d2_ref, val_ref, d2m_ref, *, blk, npad, w):
    pid = pl.program_id(0)
    xb = x_ref[...]
    sqi = jnp.sum(xb * xb, axis=1, keepdims=True)
    rowid = pid * blk + jax.lax.broadcasted_iota(jnp.int32, (blk, 1), 0)
    nch = npad // w
    for c in range(nch):
        xtc = xt_ref[:, c * w:(c + 1) * w]
        sqj = jnp.sum(xtc * xtc, axis=0, keepdims=True)
        d2c = sqi + sqj - 2.0 * jnp.dot(xb, xtc, preferred_element_type=jnp.float32)
        d2c = jnp.maximum(d2c, 0.0)
        colid = c * w + jax.lax.broadcasted_iota(jnp.int32, (blk, w), 1)
        bad = (d2c > R2) | (colid == rowid)
        d2m_ref[:, c * w:(c + 1) * w] = jnp.where(bad, _INF, d2c)

    big_i = jnp.float32(npad + 1)

    def sel_step(k, carry):
        mprev, iprev, cols, d2s, vals = carry
        mrun = jnp.full((blk, 1), _INF, jnp.float32)
        irun = jnp.full((blk, 1), big_i, jnp.float32)
        for c in range(nch):
            ch = d2m_ref[:, c * w:(c + 1) * w]
            colf = jnp.float32(c * w) + jax.lax.broadcasted_iota(
                jnp.int32, (blk, w), 1).astype(jnp.float32)
            elig = (ch > mprev) | ((ch == mprev) & (colf > iprev))
            cand = jnp.where(elig, ch, _INF)
            cmin = jnp.min(cand, axis=1, keepdims=True)
            cidx = jnp.min(jnp.where(cand == cmin, colf, big_i), axis=1,
                           keepdims=True)
            better = cmin < mrun
            mrun = jnp.where(better, cmin, mrun)
            irun = jnp.where(better, cidx, irun)
        valid = mrun <= R2
        lane = jax.lax.broadcasted_iota(jnp.int32, (blk, MAXK), 1)
        sel = lane == k
        cols = jnp.where(sel, jnp.where(valid, irun, 0.0).astype(jnp.int32), cols)
        d2s = jnp.where(sel, jnp.where(valid, mrun, 0.0), d2s)
        vals = jnp.where(sel, valid.astype(jnp.float32), vals)
        return mrun, irun, cols, d2s, vals

    init = (jnp.full((blk, 1), -1.0, jnp.float32),
            jnp.full((blk, 1), -1.0, jnp.float32),
            jnp.zeros((blk, MAXK), jnp.int32),
            jnp.zeros((blk, MAXK), jnp.float32),
            jnp.zeros((blk, MAXK), jnp.float32))
    _, _, cols, d2s, vals = jax.lax.fori_loop(0, MAXK, sel_step, init)
    col_ref[...] = cols
    d2_ref[...] = d2s
    val_ref[...] = vals


def _radius_graph_tc(x_pad, xt):
    npad = x_pad.shape[0]
    blk = 128
    w = 512
    grid = npad // blk
    return pl.pallas_call(
        functools.partial(_graph_body, blk=blk, npad=npad, w=w),
        grid=(grid,),
        in_specs=[pl.BlockSpec((blk, HID), lambda i: (i, 0)),
                  pl.BlockSpec((HID, npad), lambda i: (0, 0))],
        out_specs=[pl.BlockSpec((blk, MAXK), lambda i: (i, 0))] * 3,
        out_shape=[jax.ShapeDtypeStruct((npad, MAXK), jnp.int32),
                   jax.ShapeDtypeStruct((npad, MAXK), jnp.float32),
                   jax.ShapeDtypeStruct((npad, MAXK), jnp.float32)],
        scratch_shapes=[pltpu.VMEM((blk, npad), jnp.float32)],
    )(x_pad, xt)


def _sc_gather(table, idx_flat):
    e_pad, d = idx_flat.shape[0], table.shape[1]
    n_work = 32
    per_w = e_pad // n_work
    chunk = 64
    n_iter = per_w // chunk
    mesh = plsc.VectorSubcoreMesh(core_axis_name="c", subcore_axis_name="s")

    @functools.partial(
        pl.kernel, mesh=mesh,
        out_type=jax.ShapeDtypeStruct((e_pad, d), jnp.float32),
        scratch_types=[pltpu.VMEM((chunk,), jnp.int32),
                       pltpu.VMEM((chunk, d), jnp.float32),
                       pltpu.SemaphoreType.DMA],
    )
    def gather_k(tab_hbm, idx_hbm, out_hbm, idx_v, rows_v, sem):
        wid = jax.lax.axis_index("s") * 2 + jax.lax.axis_index("c")
        base = wid * per_w

        def body(i, carry):
            off = base + i * chunk
            pltpu.sync_copy(idx_hbm.at[pl.ds(off, chunk)], idx_v)
            pltpu.async_copy(tab_hbm.at[idx_v], rows_v, sem).wait()
            pltpu.sync_copy(rows_v, out_hbm.at[pl.ds(off, chunk)])
            return carry

        jax.lax.fori_loop(0, n_iter, body, 0)

    return gather_k(table, idx_flat)


def _edge_body(pg_ref, q_ref, feat_ref, d2_ref, val_ref, uglob_ref, gd_ref,
               e1w, e1b, e2w, e2b, g0w, g0b, g1w, g1b,
               u0aw, u0bw, u1w, u1b, u2w, u2b, out_ref, *, r):
    e = r * MAXK
    sub = jax.lax.broadcasted_iota(jnp.int32, (e, 1), 0)
    erow = sub // MAXK
    ek = sub - erow * MAXK
    expand = (erow == jax.lax.broadcasted_iota(jnp.int32, (1, r), 1)
              ).astype(jnp.float32)
    lsel = (ek == jax.lax.broadcasted_iota(jnp.int32, (1, MAXK), 1)
            ).astype(jnp.float32)
    qrep = jnp.dot(expand, q_ref[...], preferred_element_type=jnp.float32)
    d2f = jnp.sum(jnp.dot(expand, d2_ref[...],
                          preferred_element_type=jnp.float32) * lsel,
                  axis=1, keepdims=True)
    vf = jnp.sum(jnp.dot(expand, val_ref[...],
                         preferred_element_type=jnp.float32) * lsel,
                 axis=1, keepdims=True)
    pre0 = pg_ref[...] + qrep + d2f * gd_ref[...]
    a = _gelu(pre0)
    a = _gelu(jnp.dot(a, e1w[...], preferred_element_type=jnp.float32) + e1b[...])
    eh = jnp.dot(a, e2w[...], preferred_element_type=jnp.float32) + e2b[...]
    g = _gelu(jnp.dot(eh, g0w[...], preferred_element_type=jnp.float32) + g0b[...])
    g = _sigmoid(jnp.dot(g, g1w[...], preferred_element_type=jnp.float32) + g1b[...])
    msg = g * eh * vf
    seg = (jax.lax.broadcasted_iota(jnp.int32, (r, 1), 0)
           == jax.lax.broadcasted_iota(jnp.int32, (1, e), 1) // MAXK
           ).astype(jnp.float32)
    agg = jnp.dot(seg, msg, preferred_element_type=jnp.float32)
    deg = jnp.dot(seg, vf, preferred_element_type=jnp.float32)
    aggn = agg / jnp.maximum(deg, 1.0)
    up = (jnp.dot(feat_ref[...], u0aw[...], preferred_element_type=jnp.float32)
          + jnp.dot(aggn, u0bw[...], preferred_element_type=jnp.float32)
          + uglob_ref[...])
    a = _gelu(up)
    a = _gelu(jnp.dot(a, u1w[...], preferred_element_type=jnp.float32) + u1b[...])
    out_ref[...] = jnp.dot(a, u2w[...], preferred_element_type=jnp.float32) + u2b[...]


def _edge_update(pg, q, feat, d2s, vals, uglob, gd, wts):
    npad = q.shape[0]
    r = 64
    e = r * MAXK
    grid = npad // r
    row = lambda w_: pl.BlockSpec((r, w_), lambda i: (i, 0))
    full = lambda a, b: pl.BlockSpec((a, b), lambda i: (0, 0))
    (e1w, e1b, e2w, e2b, g0w, g0b, g1w, g1b,
     u0aw, u0bw, u1w, u1b, u2w, u2b) = wts
    return pl.pallas_call(
        functools.partial(_edge_body, r=r),
        grid=(grid,),
        in_specs=[pl.BlockSpec((e, MSG), lambda i: (i, 0)),
                  row(MSG), row(HID), row(MAXK), row(MAXK),
                  full(1, MSG), full(1, MSG),
                  full(MSG, MSG), full(1, MSG), full(MSG, MSG), full(1, MSG),
                  full(MSG, MSG), full(1, MSG), full(MSG, 1), full(1, 1),
                  full(HID, MSG), full(MSG, MSG), full(MSG, MSG), full(1, MSG),
                  full(MSG, HID), full(1, HID)],
        out_specs=pl.BlockSpec((r, HID), lambda i: (i, 0)),
        out_shape=jax.ShapeDtypeStruct((npad, HID), jnp.float32),
    )(pg, q, feat, d2s, vals, uglob, gd,
      e1w, e1b, e2w, e2b, g0w, g0b, g1w, g1b,
      u0aw, u0bw, u1w, u1b, u2w, u2b)


def kernel(coords, h, flow_dir, params):
    p = params
    x = coords[0]
    h0 = h[0]
    n = x.shape[0]
    npad = ((n + 2047) // 2048) * 2048

    u = flow_dir[0]
    u = u / (jnp.linalg.norm(u) + 1e-8)

    x_pad = jnp.full((npad, HID), 0.0, jnp.float32)
    x_pad = x_pad.at[:n, :3].set(x).at[n:, :3].set(100.0)
    xt = x_pad.T
    h_pad = jnp.zeros((npad, HID), jnp.float32).at[:n].set(h0)
    u_pad = jnp.zeros((1, HID), jnp.float32).at[0, :3].set(u)

    e0 = p['e0_W']
    a_w = e0[0:HID] + e0[2 * HID:3 * HID]
    b_w = e0[HID:2 * HID] - e0[2 * HID:3 * HID]
    gd = e0[3 * HID:3 * HID + 1]
    gi = e0[3 * HID + 1:3 * HID + 2] + e0[3 * HID + 3:3 * HID + 4]
    gj = e0[3 * HID + 2:3 * HID + 3] - e0[3 * HID + 3:3 * HID + 4]

    feat, q, ptab, gsum = _precompute(
        h_pad, x_pad, u_pad, p['ln_g'][None], p['ln_b'][None],
        a_w, b_w, p['e0_b'][None], gi, gj, n)

    u0 = p['u0_W']
    uglob = _global_ctx(gsum, p['gl0_W'], p['gl0_b'][None], p['gl1_W'],
                        p['gl1_b'][None], p['gl2_W'], p['gl2_b'][None],
                        u0[HID + MSG:], p['u0_b'][None], n)

    col, d2s, vals = _radius_graph_tc(x_pad, xt)

    pg = _sc_gather(ptab, col.reshape(-1))

    t = jnp.tanh(p['res_scale'])
    wts = (p['e1_W'], p['e1_b'][None], p['e2_W'], p['e2_b'][None],
           p['g0_W'], p['g0_b'][None], p['g1_W'], p['g1_b'][None],
           u0[:HID], u0[HID:HID + MSG], p['u1_W'], p['u1_b'][None],
           p['u2_W'] * t, (p['u2_b'] * t)[None])
    dh = _edge_update(pg, q, feat, d2s, vals, uglob, gd, wts)
    return dh[:n][None]
